# baseline (speedup 1.0000x reference)
import functools

import jax
import jax.numpy as jnp
from jax.experimental import pallas as pl
from jax.experimental.pallas import tpu as pltpu

_BETA = 0.25
_K = 512

_TAPS = {0: (1, 3), 1: (0, 2)}


def _off(p, t):
    return 1 - (t - 1 - p) // 2


def _dot(a, b):
    return jnp.dot(a, b, preferred_element_type=jnp.float32)


def _dot_t(a, b):
    return jax.lax.dot_general(a, b, (((0,), (0,)), ((), ())),
                               preferred_element_type=jnp.float32)


def _fused_body(p16_ref, w1_ref, b1_ref, w2_ref, b2_ref, cbt_ref, cbn_ref,
                cb_ref, wd2_ref, bd2_ref, wd1_ref, bd1_ref,
                idx_ref, xr_ref, loss_ref,
                hph_ref, qpad_ref, hdp_ref, *, scale):
    f32 = jnp.float32

    for qa in range(2):
        for qb in range(2):
            plane = jnp.maximum(
                _dot_t(p16_ref[0, qa * 2 + qb], w1_ref[...]) + b1_ref[...],
                0.0).reshape(64, 64, 64)
            a, b = (qa + 1) % 2, (qb + 1) % 2
            lo = b * 64
            hph_ref[a, 64 * a:64 * a + 1, :, lo:lo + 64] = jnp.zeros(
                (1, 65, 64), f32)
            hph_ref[a, :, 64 * b:64 * b + 1, lo:lo + 64] = jnp.zeros(
                (65, 1, 64), f32)
            hph_ref[a, qa:qa + 64, qb:qb + 64, lo:lo + 64] = plane

    acc = jnp.zeros((4096, 64), f32)
    for di in range(4):
        for dj in range(4):
            lo = (dj % 2) * 64
            win = hph_ref[di % 2, di // 2:di // 2 + 64,
                          dj // 2:dj // 2 + 64, lo:lo + 64]
            acc = acc + _dot(win.reshape(4096, 64), w2_ref[di * 4 + dj])
    flat = jnp.maximum(acc + b2_ref[...], 0.0)

    part = jnp.zeros((1, 1), f32)
    for c in range(4):
        fc = flat[c * 1024:(c + 1) * 1024]
        scores = cbn_ref[...] - 2.0 * _dot(fc, cbt_ref[...])
        m = jnp.min(scores, axis=1, keepdims=True)
        iota = jax.lax.broadcasted_iota(jnp.int32, scores.shape, 1)
        idx = jnp.min(jnp.where(scores == m, iota, _K), axis=1)
        idx_ref[0, 0:1, c * 1024:(c + 1) * 1024] = idx[None, :]
        onehot = (iota == idx[:, None]).astype(f32)
        quant = _dot(onehot, cb_ref[...])
        part = part + (jnp.sum(fc * fc) + jnp.sum(m)).reshape(1, 1)
        qpad_ref[1 + 16 * c:1 + 16 * (c + 1), 1:65, :] = quant.reshape(
            16, 64, 64)
    loss_ref[0] = part * scale

    qpad_ref[0:1, :, :] = jnp.zeros((1, 66, 64), f32)
    qpad_ref[65:66, :, :] = jnp.zeros((1, 66, 64), f32)
    qpad_ref[:, 0:1, :] = jnp.zeros((66, 1, 64), f32)
    qpad_ref[:, 65:66, :] = jnp.zeros((66, 1, 64), f32)

    for ph in range(2):
        for pw in range(2):
            acc = jnp.zeros((4096, 64), f32)
            for th in _TAPS[ph]:
                for tw in _TAPS[pw]:
                    oh, ow = _off(ph, th), _off(pw, tw)
                    win = qpad_ref[oh:oh + 64, ow:ow + 64, :]
                    acc = acc + _dot(win.reshape(4096, 64),
                                     wd2_ref[th * 4 + tw])
            plane = jnp.maximum(acc + bd2_ref[...], 0.0)
            lo = pw * 64
            hdp_ref[ph, 0:1, :, lo:lo + 64] = jnp.zeros((1, 72, 64), f32)
            hdp_ref[ph, 65:66, :, lo:lo + 64] = jnp.zeros((1, 72, 64), f32)
            hdp_ref[ph, :, 0:1, lo:lo + 64] = jnp.zeros((66, 1, 64), f32)
            hdp_ref[ph, :, 65:72, lo:lo + 64] = jnp.zeros((66, 7, 64), f32)
            hdp_ref[ph, 1:65, 1:65, lo:lo + 64] = plane.reshape(64, 64, 64)

    tpl = {}
    for p in range(2):
        for q in range(2):
            t = _dot(
                hdp_ref[p, :, :, q * 64:(q + 1) * 64].reshape(66 * 72, 64),
                wd1_ref[...])
            tpl[(p, q)] = t.reshape(66, 72, 16)
    for po_h in range(2):
        for e_h in range(2):
            for po_w in range(2):
                for e_w in range(2):
                    acc2 = jnp.zeros((64, 64), f32)
                    for th in _TAPS[po_h]:
                        for tw in _TAPS[po_w]:
                            g_h = e_h + (po_h - th + 1) // 2
                            g_w = e_w + (po_w - tw + 1) // 2
                            t = tpl[(g_h % 2, g_w % 2)]
                            oh, ow = g_h // 2 + 1, g_w // 2 + 1
                            acc2 = acc2 + t[oh:oh + 64, ow:ow + 64,
                                            th * 4 + tw]
                    xr_ref[0, po_h, e_h, po_w, e_w] = jax.nn.sigmoid(
                        acc2 + bd1_ref[0, 0])


def _fused(p16, w1, b1, w2, b2, cbt, cbn, cb, wd2, bd2, wd1, bd1, scale):
    n = p16.shape[0]
    body = functools.partial(_fused_body, scale=scale)
    return pl.pallas_call(
        body,
        grid=(n,),
        in_specs=[
            pl.BlockSpec((1, 4, 16, 4096), lambda i: (i, 0, 0, 0)),
            pl.BlockSpec((16, 64), lambda i: (0, 0)),
            pl.BlockSpec((1, 64), lambda i: (0, 0)),
            pl.BlockSpec((16, 64, 64), lambda i: (0, 0, 0)),
            pl.BlockSpec((1, 64), lambda i: (0, 0)),
            pl.BlockSpec((64, _K), lambda i: (0, 0)),
            pl.BlockSpec((1, _K), lambda i: (0, 0)),
            pl.BlockSpec((_K, 64), lambda i: (0, 0)),
            pl.BlockSpec((16, 64, 64), lambda i: (0, 0, 0)),
            pl.BlockSpec((1, 64), lambda i: (0, 0)),
            pl.BlockSpec((64, 16), lambda i: (0, 0)),
            pl.BlockSpec((1, 1), lambda i: (0, 0)),
        ],
        out_specs=[
            pl.BlockSpec((1, 1, 4096), lambda i: (i, 0, 0)),
            pl.BlockSpec((1, 2, 2, 2, 2, 64, 64),
                         lambda i: (i, 0, 0, 0, 0, 0, 0)),
            pl.BlockSpec((1, 1, 1), lambda i: (i, 0, 0)),
        ],
        out_shape=[
            jax.ShapeDtypeStruct((n, 1, 4096), jnp.int32),
            jax.ShapeDtypeStruct((n, 2, 2, 2, 2, 64, 64), jnp.float32),
            jax.ShapeDtypeStruct((n, 1, 1), jnp.float32),
        ],
        scratch_shapes=[
            pltpu.VMEM((2, 65, 65, 128), jnp.float32),
            pltpu.VMEM((66, 66, 64), jnp.float32),
            pltpu.VMEM((2, 66, 72, 128), jnp.float32),
        ],
        compiler_params=pltpu.CompilerParams(
            dimension_semantics=("arbitrary",)),
    )(p16, w1, b1, w2, b2, cbt, cbn, cb, wd2, bd2, wd1, bd1)


@jax.jit
def kernel(x, enc_w1, enc_b1, enc_w2, enc_b2, dec_w2, dec_b2, dec_w1, dec_b1,
           codebook):
    n = x.shape[0]

    xp = jnp.pad(x[:, 0], ((0, 0), (1, 3), (1, 3)))
    parities = []
    for qa in range(2):
        for qb in range(2):
            taps = []
            for di in range(4):
                for dj in range(4):
                    r, c = 2 * qa + di, 2 * qb + dj
                    taps.append(jax.lax.slice(
                        xp, (0, r, c), (n, r + 253, c + 253),
                        (1, 4, 4)).reshape(n, 4096))
            parities.append(jnp.stack(taps, axis=1))
    p16 = jnp.stack(parities, axis=1)

    w1 = enc_w1.reshape(64, 16).T
    w2 = enc_w2.reshape(64, 64, 16).transpose(2, 1, 0)
    wd2 = dec_w2.reshape(64, 64, 16).transpose(2, 0, 1)
    wd1 = dec_w1.reshape(64, 16)
    cbt = codebook.T
    cbn = jnp.sum(codebook * codebook, axis=1)[None, :]
    scale = (1.0 + _BETA) / (n * 4096 * 64.0)

    idx, xr, loss_parts = _fused(
        p16, w1, enc_b1[None, :], w2, enc_b2[None, :], cbt, cbn, codebook,
        wd2, dec_b2[None, :], wd1, dec_b1[None, :], scale)

    indices = idx.reshape(n * 4096)[:, None]
    x_recon = xr.transpose(0, 5, 2, 1, 6, 4, 3).reshape(n, 1, 256, 256)
    return (jnp.sum(loss_parts), indices, x_recon)


# speedup vs baseline: 1.0961x; 1.0961x over previous
import jax
import jax.numpy as jnp
from jax.experimental import pallas as pl
from jax.experimental.pallas import tpu as pltpu

_BETA = 0.25
_K = 512

_TAPS = {0: (1, 3), 1: (0, 2)}


def _off(p, t):
    return 1 - (t - 1 - p) // 2


def _dot(a, b):
    return jnp.dot(a, b, preferred_element_type=jnp.float32)


def _fused_body(x16_ref, w1_ref, b1_ref, w2_ref, b2_ref, cbt_ref, cbn_ref,
                cb_ref, wd2_ref, bd2_ref, wd1_ref, bd1_ref,
                idx_ref, xr_ref, loss_ref,
                hph_ref, qpad_ref, hdp_ref, *, scale):
    f32 = jnp.float32

    for qa in range(2):
        for qb in range(2):
            acc = jnp.zeros((64, 64, 64), f32)
            for di in range(4):
                for dj in range(4):
                    r, c = 2 * qa + di, 2 * qb + dj
                    win = x16_ref[0, (r % 4) * 4 + (c % 4),
                                  r // 4:r // 4 + 64, c // 4:c // 4 + 64]
                    acc = acc + win[:, :, None] * w1_ref[di * 4 + dj][None,
                                                                      None, :]
            plane = jnp.maximum(acc + b1_ref[0][None, None, :], 0.0)
            a, b = (qa + 1) % 2, (qb + 1) % 2
            hph_ref[a, b, 64 * a:64 * a + 1, :, :] = jnp.zeros((1, 65, 64),
                                                               f32)
            hph_ref[a, b, :, 64 * b:64 * b + 1, :] = jnp.zeros((65, 1, 64),
                                                               f32)
            hph_ref[a, b, qa:qa + 64, qb:qb + 64, :] = plane

    acc = jnp.zeros((4096, 64), f32)
    for di in range(4):
        for dj in range(4):
            win = hph_ref[di % 2, dj % 2,
                          di // 2:di // 2 + 64, dj // 2:dj // 2 + 64, :]
            acc = acc + _dot(win.reshape(4096, 64), w2_ref[di * 4 + dj])
    flat = jnp.maximum(acc + b2_ref[...], 0.0)

    scores = cbn_ref[...] - 2.0 * _dot(flat, cbt_ref[...])
    m = jnp.min(scores, axis=1, keepdims=True)
    iota = jax.lax.broadcasted_iota(jnp.int32, scores.shape, 1)
    idx = jnp.min(jnp.where(scores == m, iota, _K), axis=1)
    idx_ref[0] = idx[None, :]
    onehot = (iota == idx[:, None]).astype(f32)
    quant = _dot(onehot, cb_ref[...])
    part = (jnp.sum(flat * flat) + jnp.sum(m)) * scale
    loss_ref[0] = part.reshape(1, 1)

    qpad_ref[0:1, :, :] = jnp.zeros((1, 66, 64), f32)
    qpad_ref[65:66, :, :] = jnp.zeros((1, 66, 64), f32)
    qpad_ref[:, 0:1, :] = jnp.zeros((66, 1, 64), f32)
    qpad_ref[:, 65:66, :] = jnp.zeros((66, 1, 64), f32)
    qpad_ref[1:65, 1:65, :] = quant.reshape(64, 64, 64)

    for ph in range(2):
        for pw in range(2):
            acc = jnp.zeros((4096, 64), f32)
            for th in _TAPS[ph]:
                for tw in _TAPS[pw]:
                    oh, ow = _off(ph, th), _off(pw, tw)
                    win = qpad_ref[oh:oh + 64, ow:ow + 64, :]
                    acc = acc + _dot(win.reshape(4096, 64),
                                     wd2_ref[th * 4 + tw])
            plane = jnp.maximum(acc + bd2_ref[...], 0.0)
            hdp_ref[ph, pw, 0:1, :, :] = jnp.zeros((1, 72, 64), f32)
            hdp_ref[ph, pw, 65:66, :, :] = jnp.zeros((1, 72, 64), f32)
            hdp_ref[ph, pw, :, 0:1, :] = jnp.zeros((66, 1, 64), f32)
            hdp_ref[ph, pw, :, 65:72, :] = jnp.zeros((66, 7, 64), f32)
            hdp_ref[ph, pw, 1:65, 1:65, :] = plane.reshape(64, 64, 64)

    tpl = {}
    for p in range(2):
        for q in range(2):
            t = _dot(hdp_ref[p, q].reshape(66 * 72, 64), wd1_ref[...])
            tpl[(p, q)] = t.reshape(66, 72, 16)
    for po_h in range(2):
        for e_h in range(2):
            for po_w in range(2):
                for e_w in range(2):
                    acc2 = jnp.zeros((64, 64), f32)
                    for th in _TAPS[po_h]:
                        for tw in _TAPS[po_w]:
                            g_h = e_h + (po_h - th + 1) // 2
                            g_w = e_w + (po_w - tw + 1) // 2
                            t = tpl[(g_h % 2, g_w % 2)]
                            oh, ow = g_h // 2 + 1, g_w // 2 + 1
                            acc2 = acc2 + t[oh:oh + 64, ow:ow + 64,
                                            th * 4 + tw]
                    xr_ref[0, po_h, e_h, po_w, e_w] = jax.nn.sigmoid(
                        acc2 + bd1_ref[0, 0])


def _fused(x16, w1, b1, w2, b2, cbt, cbn, cb, wd2, bd2, wd1, bd1, scale):
    import functools
    n = x16.shape[0]
    body = functools.partial(_fused_body, scale=scale)
    return pl.pallas_call(
        body,
        grid=(n,),
        in_specs=[
            pl.BlockSpec((1, 16, 65, 65), lambda i: (i, 0, 0, 0)),
            pl.BlockSpec((16, 64), lambda i: (0, 0)),
            pl.BlockSpec((1, 64), lambda i: (0, 0)),
            pl.BlockSpec((16, 64, 64), lambda i: (0, 0, 0)),
            pl.BlockSpec((1, 64), lambda i: (0, 0)),
            pl.BlockSpec((64, _K), lambda i: (0, 0)),
            pl.BlockSpec((1, _K), lambda i: (0, 0)),
            pl.BlockSpec((_K, 64), lambda i: (0, 0)),
            pl.BlockSpec((16, 64, 64), lambda i: (0, 0, 0)),
            pl.BlockSpec((1, 64), lambda i: (0, 0)),
            pl.BlockSpec((64, 16), lambda i: (0, 0)),
            pl.BlockSpec((1, 1), lambda i: (0, 0)),
        ],
        out_specs=[
            pl.BlockSpec((1, 1, 4096), lambda i: (i, 0, 0)),
            pl.BlockSpec((1, 2, 2, 2, 2, 64, 64),
                         lambda i: (i, 0, 0, 0, 0, 0, 0)),
            pl.BlockSpec((1, 1, 1), lambda i: (i, 0, 0)),
        ],
        out_shape=[
            jax.ShapeDtypeStruct((n, 1, 4096), jnp.int32),
            jax.ShapeDtypeStruct((n, 2, 2, 2, 2, 64, 64), jnp.float32),
            jax.ShapeDtypeStruct((n, 1, 1), jnp.float32),
        ],
        scratch_shapes=[
            pltpu.VMEM((2, 2, 65, 65, 64), jnp.float32),
            pltpu.VMEM((66, 66, 64), jnp.float32),
            pltpu.VMEM((2, 2, 66, 72, 64), jnp.float32),
        ],
        compiler_params=pltpu.CompilerParams(
            dimension_semantics=("parallel",)),
    )(x16, w1, b1, w2, b2, cbt, cbn, cb, wd2, bd2, wd1, bd1)


@jax.jit
def kernel(x, enc_w1, enc_b1, enc_w2, enc_b2, dec_w2, dec_b2, dec_w1, dec_b1,
           codebook):
    n = x.shape[0]

    xp = jnp.pad(x[:, 0], ((0, 0), (1, 3), (1, 3)))
    x16 = xp.reshape(n, 65, 4, 65, 4).transpose(0, 2, 4, 1, 3)
    x16 = x16.reshape(n, 16, 65, 65)

    w1 = enc_w1.reshape(64, 16).T
    w2 = enc_w2.reshape(64, 64, 16).transpose(2, 1, 0)
    wd2 = dec_w2.reshape(64, 64, 16).transpose(2, 0, 1)
    wd1 = dec_w1.reshape(64, 16)
    cbt = codebook.T
    cbn = jnp.sum(codebook * codebook, axis=1)[None, :]
    scale = (1.0 + _BETA) / (n * 4096 * 64.0)

    idx, xr, loss_parts = _fused(
        x16, w1, enc_b1[None, :], w2, enc_b2[None, :], cbt, cbn, codebook,
        wd2, dec_b2[None, :], wd1, dec_b1[None, :], scale)

    indices = idx.reshape(n * 4096)[:, None]
    x_recon = xr.transpose(0, 5, 2, 1, 6, 4, 3).reshape(n, 1, 256, 256)
    return (jnp.sum(loss_parts), indices, x_recon)
